# baseline (speedup 1.0000x reference)
import functools

import jax
import jax.numpy as jnp
from jax.experimental import pallas as pl

BS, T, DIM, ND, NC = 1024, 16, 512, 365, 24
NDP, NCP, NCLS = 384, 32, 128
BLK = 256

_HI = jax.lax.Precision.HIGHEST


def _dot(a, b):
    return jax.lax.dot_general(a, b, (((1,), (0,)), ((), ())),
                               preferred_element_type=jnp.float32,
                               precision=_HI)


def _dot_t(a, b):
    return jax.lax.dot_general(a, b, (((0,), (0,)), ((), ())),
                               preferred_element_type=jnp.float32,
                               precision=_HI)


def _prelu(x, a):
    return jnp.where(x >= 0, x, a * x)


def _mean_body(f_ref, o_ref):
    o_ref[:] = jnp.mean(f_ref[:], axis=1)


def _main_body(video_ref, vd_ref, dv_ref, sub_ref, ev_ref, dc_ref, gt_ref,
               Wc2d_ref, bc2d_ref, ac2d_ref,
               Wv2d_ref, bv2d_ref, av2d_ref,
               Wd2v_ref, bd2v_ref, ad2v_ref,
               Wd2v2_ref, bd2v2_ref, ad2v2_ref,
               Wsa_ref, bsa_ref, qsa_ref,
               Wfc1_ref, Wfc2_ref, Wfc3_ref, bfc_ref,
               Wcls_ref, bcls_ref,
               loss_ref, idx_ref):
    v_emb = video_ref[:]

    vW = _dot(v_emb, Wv2d_ref[:])
    v2d = _prelu(_dot_t(dv_ref[:], vW) + bv2d_ref[:], av2d_ref[0, 0])

    eW = _dot(ev_ref[:], Wc2d_ref[:])
    c2d = _prelu(_dot(dc_ref[:], eW) + bc2d_ref[:], ac2d_ref[0, 0])

    qsa = qsa_ref[:]
    mask = jax.lax.broadcasted_iota(jnp.int32, (NDP, DIM // 4), 0) < ND
    hc = jnp.tanh(_dot(c2d, Wsa_ref[:]) + bsa_ref[:])
    hv = jnp.tanh(_dot(v2d, Wsa_ref[:]) + bsa_ref[:])
    sc = jnp.sum(jnp.where(mask, hc * qsa, 0.0)) / ND
    sv = jnp.sum(jnp.where(mask, hv * qsa, 0.0)) / ND
    m = jnp.maximum(sc, sv)
    e0, e1 = jnp.exp(sc - m), jnp.exp(sv - m)
    w0 = e0 / (e0 + e1)
    w1 = e1 / (e0 + e1)
    att = w0 * c2d + w1 * v2d

    vd = vd_ref[:]
    sW = _dot(sub_ref[:], Wd2v_ref[:])
    d2v = _prelu(_dot(vd, sW) + bd2v_ref[:], ad2v_ref[0, 0])
    aW2 = _dot(att, Wd2v2_ref[:])
    d2v2 = _prelu(_dot(vd, aW2) + bd2v2_ref[:], ad2v2_ref[0, 0])

    vc = (_dot(d2v2, Wfc1_ref[:]) + _dot(d2v, Wfc2_ref[:])
          + _dot(v_emb, Wfc3_ref[:]) + bfc_ref[:])
    preds = _dot(vc, Wcls_ref[:]) + bcls_ref[:]

    mx = jnp.max(preds, axis=1, keepdims=True)
    z = preds - mx
    lse = jnp.log(jnp.sum(jnp.exp(z), axis=1, keepdims=True))
    cls_ids = jax.lax.broadcasted_iota(jnp.int32, (BS, NCLS), 1)
    z_gt = jnp.sum(jnp.where(cls_ids == gt_ref[:], z, 0.0), axis=1,
                   keepdims=True)
    loss_ref[:] = (-jnp.sum(z_gt - lse, keepdims=True) / BS).reshape(1, 1)
    idx_ref[:] = jnp.min(jnp.where(preds == mx, cls_ids, NCLS), axis=1,
                         keepdims=True)


@functools.partial(jax.jit, static_argnames=())
def kernel(frame_emb, cd_adj, dc_adj, vd_adj, dv_adj, subevent, event,
           logit_scale, ground_truth, W_c2d, b_c2d, a_c2d, W_v2d, b_v2d,
           a_v2d, W_d2v, b_d2v, a_d2v, W_d2v2, b_d2v2, a_d2v2, W_sa, b_sa,
           q_sa, W_fc, b_fc, W_cls, b_cls):
    del cd_adj, logit_scale

    f32 = jnp.float32
    pad_nd = NDP - ND
    vd_p = jnp.pad(vd_adj, ((0, 0), (0, pad_nd)))
    dv_p = jnp.pad(dv_adj, ((0, 0), (0, pad_nd)))
    sub_p = jnp.pad(subevent, ((0, pad_nd), (0, 0)))
    dc_p = jnp.pad(dc_adj, ((0, pad_nd), (0, NCP - NC)))
    ev_p = jnp.pad(event, ((0, NCP - NC), (0, 0)))
    Wcls_p = jnp.pad(W_cls, ((0, 0), (0, NCLS - NC)))
    bcls_p = jnp.concatenate(
        [b_cls, jnp.full((NCLS - NC,), -1e30, f32)]).reshape(1, NCLS)
    Wfc1, Wfc2, Wfc3 = W_fc[:DIM], W_fc[DIM:2 * DIM], W_fc[2 * DIM:]
    gt2 = ground_truth.reshape(BS, 1)
    s = lambda x: x.reshape(1, 1)
    r = lambda x: x.reshape(1, -1)

    video_emb = pl.pallas_call(
        _mean_body,
        grid=(BS // BLK,),
        in_specs=[pl.BlockSpec((BLK, T, DIM), lambda i: (i, 0, 0))],
        out_specs=pl.BlockSpec((BLK, DIM), lambda i: (i, 0)),
        out_shape=jax.ShapeDtypeStruct((BS, DIM), f32),
    )(frame_emb)

    loss2, idx = pl.pallas_call(
        _main_body,
        out_shape=(jax.ShapeDtypeStruct((1, 1), f32),
                   jax.ShapeDtypeStruct((BS, 1), jnp.int32)),
    )(video_emb, vd_p, dv_p, sub_p, ev_p, dc_p, gt2,
      W_c2d, r(b_c2d), s(a_c2d),
      W_v2d, r(b_v2d), s(a_v2d),
      W_d2v, r(b_d2v), s(a_d2v),
      W_d2v2, r(b_d2v2), s(a_d2v2),
      W_sa, b_sa, q_sa,
      Wfc1, Wfc2, Wfc3, r(b_fc),
      Wcls_p, bcls_p)

    return loss2[0, 0], idx


# speedup vs baseline: 1.5860x; 1.5860x over previous
import functools

import jax
import jax.numpy as jnp
from jax.experimental import pallas as pl

BS, T, DIM, ND, NC = 1024, 16, 512, 365, 24
NDP, NCP, NCLS = 384, 32, 128
BLK = 256

_HI = jax.lax.Precision.DEFAULT


def _dot(a, b):
    return jax.lax.dot_general(a, b, (((1,), (0,)), ((), ())),
                               preferred_element_type=jnp.float32,
                               precision=_HI)


def _dot_t(a, b):
    return jax.lax.dot_general(a, b, (((0,), (0,)), ((), ())),
                               preferred_element_type=jnp.float32,
                               precision=_HI)


def _prelu(x, a):
    return jnp.where(x >= 0, x, a * x)


def _mean_body(f_ref, o_ref):
    o_ref[:] = jnp.mean(f_ref[:], axis=1)


def _main_body(video_ref, vd_ref, dv_ref, sub_ref, ev_ref, dc_ref, gt_ref,
               Wc2d_ref, bc2d_ref, ac2d_ref,
               Wv2d_ref, bv2d_ref, av2d_ref,
               Wd2v_ref, bd2v_ref, ad2v_ref,
               Wd2v2_ref, bd2v2_ref, ad2v2_ref,
               Wsa_ref, bsa_ref, qsa_ref,
               Wfc1_ref, Wfc2_ref, Wfc3_ref, bfc_ref,
               Wcls_ref, bcls_ref,
               loss_ref, idx_ref):
    v_emb = video_ref[:]

    vW = _dot(v_emb, Wv2d_ref[:])
    v2d = _prelu(_dot_t(dv_ref[:], vW) + bv2d_ref[:], av2d_ref[0, 0])

    eW = _dot(ev_ref[:], Wc2d_ref[:])
    c2d = _prelu(_dot(dc_ref[:], eW) + bc2d_ref[:], ac2d_ref[0, 0])

    qsa = qsa_ref[:]
    mask = jax.lax.broadcasted_iota(jnp.int32, (NDP, DIM // 4), 0) < ND
    hc = jnp.tanh(_dot(c2d, Wsa_ref[:]) + bsa_ref[:])
    hv = jnp.tanh(_dot(v2d, Wsa_ref[:]) + bsa_ref[:])
    sc = jnp.sum(jnp.where(mask, hc * qsa, 0.0)) / ND
    sv = jnp.sum(jnp.where(mask, hv * qsa, 0.0)) / ND
    m = jnp.maximum(sc, sv)
    e0, e1 = jnp.exp(sc - m), jnp.exp(sv - m)
    w0 = e0 / (e0 + e1)
    w1 = e1 / (e0 + e1)
    att = w0 * c2d + w1 * v2d

    vd = vd_ref[:]
    sW = _dot(sub_ref[:], Wd2v_ref[:])
    d2v = _prelu(_dot(vd, sW) + bd2v_ref[:], ad2v_ref[0, 0])
    aW2 = _dot(att, Wd2v2_ref[:])
    d2v2 = _prelu(_dot(vd, aW2) + bd2v2_ref[:], ad2v2_ref[0, 0])

    vc = (_dot(d2v2, Wfc1_ref[:]) + _dot(d2v, Wfc2_ref[:])
          + _dot(v_emb, Wfc3_ref[:]) + bfc_ref[:])
    preds = _dot(vc, Wcls_ref[:]) + bcls_ref[:]

    mx = jnp.max(preds, axis=1, keepdims=True)
    z = preds - mx
    lse = jnp.log(jnp.sum(jnp.exp(z), axis=1, keepdims=True))
    cls_ids = jax.lax.broadcasted_iota(jnp.int32, (BS, NCLS), 1)
    z_gt = jnp.sum(jnp.where(cls_ids == gt_ref[:], z, 0.0), axis=1,
                   keepdims=True)
    loss_ref[:] = (-jnp.sum(z_gt - lse, keepdims=True) / BS).reshape(1, 1)
    idx_ref[:] = jnp.min(jnp.where(preds == mx, cls_ids, NCLS), axis=1,
                         keepdims=True)


@functools.partial(jax.jit, static_argnames=())
def kernel(frame_emb, cd_adj, dc_adj, vd_adj, dv_adj, subevent, event,
           logit_scale, ground_truth, W_c2d, b_c2d, a_c2d, W_v2d, b_v2d,
           a_v2d, W_d2v, b_d2v, a_d2v, W_d2v2, b_d2v2, a_d2v2, W_sa, b_sa,
           q_sa, W_fc, b_fc, W_cls, b_cls):
    del cd_adj, logit_scale

    f32 = jnp.float32
    pad_nd = NDP - ND
    vd_p = jnp.pad(vd_adj, ((0, 0), (0, pad_nd)))
    dv_p = jnp.pad(dv_adj, ((0, 0), (0, pad_nd)))
    sub_p = jnp.pad(subevent, ((0, pad_nd), (0, 0)))
    dc_p = jnp.pad(dc_adj, ((0, pad_nd), (0, NCP - NC)))
    ev_p = jnp.pad(event, ((0, NCP - NC), (0, 0)))
    Wcls_p = jnp.pad(W_cls, ((0, 0), (0, NCLS - NC)))
    bcls_p = jnp.concatenate(
        [b_cls, jnp.full((NCLS - NC,), -1e30, f32)]).reshape(1, NCLS)
    Wfc1, Wfc2, Wfc3 = W_fc[:DIM], W_fc[DIM:2 * DIM], W_fc[2 * DIM:]
    gt2 = ground_truth.reshape(BS, 1)
    s = lambda x: x.reshape(1, 1)
    r = lambda x: x.reshape(1, -1)

    video_emb = pl.pallas_call(
        _mean_body,
        grid=(BS // BLK,),
        in_specs=[pl.BlockSpec((BLK, T, DIM), lambda i: (i, 0, 0))],
        out_specs=pl.BlockSpec((BLK, DIM), lambda i: (i, 0)),
        out_shape=jax.ShapeDtypeStruct((BS, DIM), f32),
    )(frame_emb)

    loss2, idx = pl.pallas_call(
        _main_body,
        out_shape=(jax.ShapeDtypeStruct((1, 1), f32),
                   jax.ShapeDtypeStruct((BS, 1), jnp.int32)),
    )(video_emb, vd_p, dv_p, sub_p, ev_p, dc_p, gt2,
      W_c2d, r(b_c2d), s(a_c2d),
      W_v2d, r(b_v2d), s(a_v2d),
      W_d2v, r(b_d2v), s(a_d2v),
      W_d2v2, r(b_d2v2), s(a_d2v2),
      W_sa, b_sa, q_sa,
      Wfc1, Wfc2, Wfc3, r(b_fc),
      Wcls_p, bcls_p)

    return loss2[0, 0], idx
